# Initial kernel scaffold; baseline (speedup 1.0000x reference)
import jax
import jax.numpy as jnp
from jax.experimental import pallas as pl


def kernel(pred, gt):
    raise NotImplementedError("write your pallas kernel here")


# speedup vs baseline: 1.9568x; 1.9568x over previous
import jax
import jax.numpy as jnp
from jax.experimental import pallas as pl
from jax.experimental.pallas import tpu as pltpu

_LAMB_COORD = 5.0
_LAMB_NOOBJ = 0.5
_B, _C, _S2 = 2048, 30, 49
_BB = 128


def _body(p_ref, g_ref, out_ref):
    i = pl.program_id(0)
    p = p_ref[...]
    g = g_ref[...]
    obj = (g[:, 4:5, :] == 1.0).astype(jnp.float32)
    chan = jax.lax.broadcasted_iota(jnp.int32, (1, _C, 1), 1)
    is_wh = (chan == 2) | (chan == 3) | (chan == 7) | (chan == 8)
    is_xy = (chan == 0) | (chan == 1) | (chan == 5) | (chan == 6)
    is_conf = (chan == 4) | (chan == 9)
    w_b = jnp.where(is_xy | is_wh, _LAMB_COORD,
                    jnp.where(is_conf, 1.0 - _LAMB_NOOBJ, 1.0))
    w_a = jnp.where(is_conf, _LAMB_NOOBJ, 0.0)
    d = p - g
    sq = jnp.where(is_wh, p + g - 2.0 * jnp.sqrt(p * g), d * d)
    part = jnp.sum((w_a + w_b * obj) * sq)

    @pl.when(i == 0)
    def _():
        out_ref[0, 0] = 0.0

    out_ref[0, 0] += part


def kernel(pred, gt):
    b = pred.shape[0]
    p3 = pred.reshape(_B, _C, _S2)
    g3 = gt.reshape(_B, _C, _S2)
    out = pl.pallas_call(
        _body,
        grid=(_B // _BB,),
        in_specs=[
            pl.BlockSpec((_BB, _C, _S2), lambda i: (i, 0, 0)),
            pl.BlockSpec((_BB, _C, _S2), lambda i: (i, 0, 0)),
        ],
        out_specs=pl.BlockSpec(memory_space=pltpu.SMEM),
        out_shape=jax.ShapeDtypeStruct((1, 1), jnp.float32),
    )(p3, g3)
    return out[0, 0] / b
